# Initial kernel scaffold; baseline (speedup 1.0000x reference)
import jax
import jax.numpy as jnp
from jax.experimental import pallas as pl


def kernel(x, h, c, edge_index, W_iou, U_iou, b_iou, U_f_w, U_f_b, W_f, b_f):
    raise NotImplementedError("write your pallas kernel here")


# speedup vs baseline: 9.6663x; 9.6663x over previous
import functools

import jax
import jax.numpy as jnp
from jax import lax
from jax.experimental import pallas as pl
from jax.experimental.pallas import tpu as pltpu
from jax.experimental.pallas import tpu_sc as plsc

N = 10000
E = 320000
D = 128
NUM_SUBCORES = 16
CHUNK = 125
EDGES_PER_TILE = E // NUM_SUBCORES
NCHUNK = EDGES_PER_TILE // CHUNK
SUBCH = 16
NBLOCK = NCHUNK // SUBCH
N_PAD = 10240
NPT = N_PAD // NUM_SUBCORES


def _seg_body(h_hbm, c_hbm, src_hbm, dst_hbm, zeros_hbm, hsum_out, csum_out,
              acc, src_buf, dst_buf, rows0, rows1, sem0, sem1):
    cid = lax.axis_index("c")
    sid = lax.axis_index("s")
    row0 = sid * NPT

    pltpu.sync_copy(zeros_hbm, acc.at[pl.ds(row0, NPT)])
    plsc.subcore_barrier()

    def accumulate(table_ref):
        def outer(b, _):
            pltpu.sync_copy(src_hbm.at[sid, pl.ds(b * SUBCH, SUBCH)], src_buf)
            pltpu.sync_copy(dst_hbm.at[sid, pl.ds(b * SUBCH, SUBCH)], dst_buf)

            def body(j2, _):
                j = j2 * 2
                d0 = pltpu.async_copy(table_ref.at[src_buf.at[j]], rows0, sem0)
                d1 = pltpu.async_copy(table_ref.at[src_buf.at[j + 1]], rows1,
                                      sem1)
                d0.wait()
                pltpu.sync_copy(rows0, acc.at[dst_buf.at[j]], add=True)
                d1.wait()
                pltpu.sync_copy(rows1, acc.at[dst_buf.at[j + 1]], add=True)
                return _
            lax.fori_loop(0, SUBCH // 2, body, None)
            return _
        lax.fori_loop(0, NBLOCK, outer, None)

    @pl.when(cid == 0)
    def _():
        accumulate(h_hbm)

    @pl.when(cid == 1)
    def _():
        accumulate(c_hbm)

    plsc.subcore_barrier()

    @pl.when(cid == 0)
    def _():
        pltpu.sync_copy(acc.at[pl.ds(row0, NPT)], hsum_out.at[pl.ds(row0, NPT)])

    @pl.when(cid == 1)
    def _():
        pltpu.sync_copy(acc.at[pl.ds(row0, NPT)], csum_out.at[pl.ds(row0, NPT)])


_segment_sums = functools.partial(
    pl.kernel,
    out_type=(
        jax.ShapeDtypeStruct((N_PAD, D), jnp.float32),
        jax.ShapeDtypeStruct((N_PAD, D), jnp.float32),
    ),
    mesh=plsc.VectorSubcoreMesh(core_axis_name="c", subcore_axis_name="s"),
    scratch_types=(
        pltpu.VMEM_SHARED((N_PAD, D), jnp.float32),
        pltpu.VMEM((SUBCH, CHUNK), jnp.int32),
        pltpu.VMEM((SUBCH, CHUNK), jnp.int32),
        pltpu.VMEM((CHUNK, D), jnp.float32),
        pltpu.VMEM((CHUNK, D), jnp.float32),
        pltpu.SemaphoreType.DMA,
        pltpu.SemaphoreType.DMA,
    ),
)(_seg_body)


_BLK = 1000


def _gates_body(x_r, h_r, hs_r, cs_r, wiou_r, uiou_r, biou_r, ufw_r, ufb_r,
                wf_r, bf_r, hn_r, cn_r):
    xb = x_r[...]
    hb = h_r[...]
    f = jax.nn.sigmoid(
        jnp.dot(xb, wf_r[...], preferred_element_type=jnp.float32)
        + jnp.dot(hb, ufw_r[...], preferred_element_type=jnp.float32)
        + ufb_r[...] + bf_r[...])
    iou = (jnp.dot(xb, wiou_r[...], preferred_element_type=jnp.float32)
           + jnp.dot(hs_r[...], uiou_r[...], preferred_element_type=jnp.float32)
           + biou_r[...])
    i = jax.nn.sigmoid(iou[:, :D])
    o = jax.nn.sigmoid(iou[:, D:2 * D])
    u = jnp.tanh(iou[:, 2 * D:])
    c_new = i * u + f * cs_r[...]
    cn_r[...] = c_new
    hn_r[...] = o * jnp.tanh(c_new)


def _gates_tc(x, h, h_sum, c_sum, W_iou, U_iou, b_iou, U_f_w, U_f_b2, W_f, b_f):
    grid = (N // _BLK,)
    row_spec = pl.BlockSpec((_BLK, D), lambda i: (i, 0))
    full = lambda shape: pl.BlockSpec(shape, lambda i: (0,) * len(shape))
    return pl.pallas_call(
        _gates_body,
        grid=grid,
        in_specs=[
            row_spec, row_spec, row_spec, row_spec,
            full((D, 3 * D)), full((D, 3 * D)), full((1, 3 * D)),
            full((D, D)), full((1, D)), full((D, D)), full((1, D)),
        ],
        out_specs=[row_spec, row_spec],
        out_shape=[
            jax.ShapeDtypeStruct((N, D), jnp.float32),
            jax.ShapeDtypeStruct((N, D), jnp.float32),
        ],
    )(x, h, h_sum, c_sum, W_iou, U_iou, b_iou, U_f_w, U_f_b2, W_f, b_f)


@jax.jit
def kernel(x, h, c, edge_index, W_iou, U_iou, b_iou, U_f_w, U_f_b, W_f, b_f):
    src3d = edge_index[0].reshape(NUM_SUBCORES, NCHUNK, CHUNK)
    dst3d = edge_index[1].reshape(NUM_SUBCORES, NCHUNK, CHUNK)
    zeros = jnp.zeros((NPT, D), dtype=jnp.float32)
    h_sum, c_sum = _segment_sums(h, c, src3d, dst3d, zeros)
    return _gates_tc(x, h, h_sum, c_sum, W_iou, U_iou, b_iou, U_f_w,
                     U_f_b.reshape(1, D), W_f, b_f)
